# Initial kernel scaffold; baseline (speedup 1.0000x reference)
import jax
import jax.numpy as jnp
from jax.experimental import pallas as pl


def kernel(x, tw1, tw2, tw3, tf1w, tf1g, tf1b, tf2w, tf2bias, tf2g, tf2b, tf3w, tf3bias, w1, w2, w3, w4, w5, bn5g, bn5b):
    raise NotImplementedError("write your pallas kernel here")


# speedup vs baseline: 3.1688x; 3.1688x over previous
import functools

import jax
import jax.numpy as jnp
from jax import lax
from jax.experimental import pallas as pl
from jax.experimental.pallas import tpu as pltpu
from jax.experimental.pallas import tpu_sc as plsc

N = 1024
KNN = 20
B = 4
NK = N * KNN
BN = B * N
BNK = B * N * KNN
EPS = 1e-5
F32 = jnp.float32
BF16 = jnp.bfloat16
I32 = jnp.int32

NUM_WORKERS = 32


def _lrelu(z):
    return jnp.maximum(z, 0.2 * z)


def _bfdot(a, b):
    return jnp.dot(a.astype(BF16), b.astype(BF16), preferred_element_type=F32)


def _pd(x):
    g = _bfdot(x, x.T)
    d = jnp.sum(x * x, axis=1, keepdims=True)
    return 2.0 * g - d - d.T


def _topk_idx(pd, base):
    p = pd.reshape(N, 8, 128)
    li3 = lax.broadcasted_iota(I32, (N, 8, 128), 2)
    ji3 = lax.broadcasted_iota(I32, (N, 8, 128), 1) * 128 + li3
    g = jnp.max(p, axis=1)
    cols = []
    for _ in range(KNN):
        m = jnp.max(g, axis=1, keepdims=True)
        sel = jnp.min(jnp.where(p == m[:, :, None], ji3, N * N),
                      axis=(1, 2))
        cols.append(sel[:, None] + base)
        p = jnp.where(ji3 == sel[:, None, None], -jnp.inf, p)
        g = jnp.max(p, axis=1)
    return jnp.concatenate(cols, axis=1)


def _knn0_body(xt_ref, idx_ref):
    b = pl.program_id(0)
    idx_ref[0] = _topk_idx(_pd(xt_ref[0]), b * N)


def _knn0_call(xtpad):
    return pl.pallas_call(
        _knn0_body,
        grid=(B,),
        in_specs=[pl.BlockSpec((1, N, 16), lambda b: (b, 0, 0))],
        out_specs=pl.BlockSpec((1, N, KNN), lambda b: (b, 0, 0)),
        out_shape=jax.ShapeDtypeStruct((B, N, KNN), I32),
    )(xtpad)


def _conv_in_lrelu(x, w):
    y = jnp.einsum('oc,bcnk->bonk', w, x)
    mu = jnp.mean(y, axis=(2, 3), keepdims=True)
    var = jnp.var(y, axis=(2, 3), keepdims=True)
    return jax.nn.leaky_relu((y - mu) / jnp.sqrt(var + EPS), 0.2)


def _fcb(x, w, b, g, beta):
    x = x / jnp.linalg.norm(x, axis=1, keepdims=True)
    y = x @ w.T
    if b is not None:
        y = y + b
    mu = jnp.mean(y, axis=-1, keepdims=True)
    var = jnp.var(y, axis=-1, keepdims=True)
    return jax.nn.leaky_relu((y - mu) / jnp.sqrt(var + EPS) * g + beta, 0.2)


def _transform_head(xg0, xt, x, tw1, tw2, tw3, tf1w, tf1g, tf1b,
                    tf2w, tf2bias, tf2g, tf2b, tf3w, tf3bias):
    feat = jnp.transpose(xg0.reshape(B, KNN, N, 16)[..., :3], (0, 2, 1, 3))
    center = jnp.broadcast_to(xt[:, :, None, :3], (B, N, KNN, 3))
    x0 = jnp.transpose(
        jnp.concatenate([feat - center, center], axis=3), (0, 3, 1, 2))
    t = _conv_in_lrelu(x0, tw1)
    t = _conv_in_lrelu(t, tw2)
    t = jnp.max(t, axis=-1)[..., None]
    t = _conv_in_lrelu(t, tw3)
    t = jnp.max(t, axis=2).reshape(B, -1)
    t = _fcb(t, tf1w, None, tf1g, tf1b)
    t = _fcb(t, tf2w, tf2bias, tf2g, tf2b)
    t = t @ tf3w.T + tf3bias
    t = (t + jnp.eye(3, dtype=x.dtype).reshape(1, 9)).reshape(B, 3, 3)
    return jnp.einsum('bij,bjn->bin', t, x)


def _block_body(xg_ref, x_ref, w_ref, xo_ref, idx_ref=None, *, cn, last):
    b = pl.program_id(0)
    x = x_ref[0]
    xg = xg_ref[0]
    w = w_ref[...].astype(BF16)
    ss = jnp.zeros((1, cn), F32)
    qs = jnp.zeros((1, cn), F32)
    mx = jnp.full((N, cn), -jnp.inf, F32)
    for k in range(KNN):
        cat = jnp.concatenate([xg[k * N:(k + 1) * N] - x, x], axis=1)
        y = jnp.dot(cat.astype(BF16), w, preferred_element_type=F32)
        ss = ss + jnp.sum(y, axis=0, keepdims=True)
        qs = qs + jnp.sum(y * y, axis=0, keepdims=True)
        mx = jnp.maximum(mx, y)
    mu = ss / NK
    var = qs / NK - mu * mu
    xo = _lrelu((mx - mu) / jnp.sqrt(var + EPS))
    xo_ref[0] = xo
    if not last:
        idx_ref[0] = _topk_idx(_pd(xo), b * N)


def _block_call(xg, x, wcat, cp, cn, last):
    bs = lambda d: pl.BlockSpec((1, N, d), lambda b: (b, 0, 0))
    out_specs = [bs(cn)]
    out_shape = [jax.ShapeDtypeStruct((B, N, cn), F32)]
    if not last:
        out_specs.append(pl.BlockSpec((1, N, KNN), lambda b: (b, 0, 0)))
        out_shape.append(jax.ShapeDtypeStruct((B, N, KNN), I32))
    res = pl.pallas_call(
        functools.partial(_block_body, cn=cn, last=last),
        grid=(B,),
        in_specs=[pl.BlockSpec((1, NK, cp), lambda b: (b, 0, 0)), bs(cp),
                  pl.BlockSpec((2 * cp, cn), lambda b: (0, 0))],
        out_specs=out_specs,
        out_shape=out_shape,
    )(xg, x, wcat)
    return res if not last else (res[0], None)


def _final_body(x1_ref, x2_ref, x3_ref, x4_ref, w5t_ref, g_ref, beta_ref,
                out_ref):
    w5t = w5t_ref[...].astype(BF16)
    ys = []
    ssum = jnp.zeros((1, 1024), F32)
    for b in range(B):
        xcat = jnp.concatenate(
            [x1_ref[b], x2_ref[b], x3_ref[b], x4_ref[b]], axis=1)
        y = jnp.dot(xcat.astype(BF16), w5t, preferred_element_type=F32)
        ys.append(y)
        ssum = ssum + jnp.sum(y, axis=0, keepdims=True)
    mu = ssum / (B * N)
    qsum = jnp.zeros((1, 1024), F32)
    for b in range(B):
        cb = ys[b] - mu
        qsum = qsum + jnp.sum(cb * cb, axis=0, keepdims=True)
    var = qsum / (B * N)
    sd = jnp.sqrt(var + EPS)
    rows = []
    for b in range(B):
        yn = _lrelu((ys[b] - mu) / sd * g_ref[...] + beta_ref[...])
        rows.append(jnp.concatenate(
            [jnp.max(yn, axis=0, keepdims=True),
             jnp.sum(yn, axis=0, keepdims=True) / N], axis=1))
    out_ref[...] = jnp.concatenate(rows, axis=0)


def _final_call(x1, x2, x3, x4, w5t, bn5g, bn5b):
    full = lambda *s: pl.BlockSpec(s, lambda: (0,) * len(s))
    return pl.pallas_call(
        _final_body,
        in_specs=[full(B, N, 64), full(B, N, 64), full(B, N, 128),
                  full(B, N, 256), full(512, 1024), full(1, 1024),
                  full(1, 1024)],
        out_specs=full(B, 2048),
        out_shape=jax.ShapeDtypeStruct((B, 2048), F32),
    )(x1, x2, x3, x4, w5t, bn5g, bn5b)


def _sc_gather_rows(table, idxflat, c):
    rows_pw = BNK // NUM_WORKERS
    nch = rows_pw // 80
    mesh = plsc.VectorSubcoreMesh(core_axis_name="c", subcore_axis_name="s")

    @functools.partial(
        pl.kernel, mesh=mesh,
        compiler_params=pltpu.CompilerParams(use_tc_tiling_on_sc=False),
        out_type=jax.ShapeDtypeStruct((BNK, c), F32),
        scratch_types=[
            pltpu.VMEM((80,), I32),
            pltpu.VMEM((80, c), F32),
            pltpu.SemaphoreType.DMA,
        ],
    )
    def k(table_hbm, idx_hbm, out_hbm, idx_v, rows_v, sem):
        wid = lax.axis_index("s") * 2 + lax.axis_index("c")

        def body(ch, _):
            base = wid * rows_pw + ch * 80
            pltpu.sync_copy(idx_hbm.at[pl.ds(base, 80)], idx_v)
            pltpu.async_copy(table_hbm.at[idx_v], rows_v, sem).wait()
            pltpu.sync_copy(rows_v, out_hbm.at[pl.ds(base, 80)])
            return 0

        lax.fori_loop(0, nch, body, 0)

    return k(table, idxflat)


def _gather_kmajor(table, idx, c):
    idxk = jnp.transpose(idx, (0, 2, 1)).reshape(-1)
    return _sc_gather_rows(table.reshape(BN, c), idxk, c).reshape(B, NK, c)


def kernel(x, tw1, tw2, tw3, tf1w, tf1g, tf1b, tf2w, tf2bias, tf2g, tf2b,
           tf3w, tf3bias, w1, w2, w3, w4, w5, bn5g, bn5b):
    xt = jnp.transpose(x, (0, 2, 1))
    xtpad = jnp.concatenate(
        [xt, jnp.zeros((B, N, 13), F32)], axis=2)

    def wcat6(w):
        z = jnp.zeros((13, w.shape[0]), F32)
        return jnp.concatenate([w[:, :3].T, z, w[:, 3:].T, z], axis=0)

    idx0 = _knn0_call(xtpad)
    xg0 = _gather_kmajor(xtpad, idx0, 16)

    xp = _transform_head(xg0, xtpad, x, tw1, tw2, tw3, tf1w, tf1g, tf1b,
                         tf2w, tf2bias, tf2g, tf2b, tf3w, tf3bias)
    xtp = jnp.concatenate(
        [jnp.transpose(xp, (0, 2, 1)), jnp.zeros((B, N, 13), F32)], axis=2)
    idx1 = _knn0_call(xtp)

    xg1 = _gather_kmajor(xtp, idx1, 16)
    x1, idx2 = _block_call(xg1, xtp, wcat6(w1), 16, 64, False)

    xg2 = _gather_kmajor(x1, idx2, 64)
    x2, idx3 = _block_call(
        xg2, x1, jnp.concatenate([w2[:, :64].T, w2[:, 64:].T], axis=0),
        64, 64, False)

    xg3 = _gather_kmajor(x2, idx3, 64)
    x3, idx4 = _block_call(
        xg3, x2, jnp.concatenate([w3[:, :64].T, w3[:, 64:].T], axis=0),
        64, 128, False)

    xg4 = _gather_kmajor(x3, idx4, 128)
    x4, _ = _block_call(
        xg4, x3, jnp.concatenate([w4[:, :128].T, w4[:, 128:].T], axis=0),
        128, 256, True)

    return _final_call(x1, x2, x3, x4, w5.T, bn5g[None], bn5b[None])
